# baseline (speedup 1.0000x reference)
import functools

import jax
import jax.numpy as jnp
from jax import lax
from jax.experimental import pallas as pl
from jax.experimental.pallas import tpu as pltpu
from jax.experimental.pallas import tpu_sc as plsc

N = 10000
E = 320000
D = 128
DA = 144
NP = 10240

NC = 2
NS = 16
NW = NC * NS
CHUNK = 64
NCHUNK = 158
EPT = NCHUNK * CHUNK
EP = NW * EPT
ZPT = 8
ZC = NP // NS // ZPT


def _sc_aggregate(faug, src, dst):
    mesh = plsc.VectorSubcoreMesh(core_axis_name="c", subcore_axis_name="s")

    @functools.partial(
        pl.kernel,
        mesh=mesh,
        compiler_params=pltpu.CompilerParams(use_tc_tiling_on_sc=False),
        out_type=jax.ShapeDtypeStruct((NC, NP, DA), jnp.float32),
        scratch_types=[
            pltpu.VMEM((CHUNK,), jnp.int32),
            pltpu.VMEM((CHUNK,), jnp.int32),
            pltpu.VMEM((CHUNK,), jnp.int32),
            pltpu.VMEM((CHUNK,), jnp.int32),
            pltpu.VMEM((CHUNK, DA), jnp.float32),
            pltpu.VMEM((CHUNK, DA), jnp.float32),
            pltpu.VMEM_SHARED((NP, DA), jnp.float32),
            pltpu.SemaphoreType.DMA,
            pltpu.SemaphoreType.DMA,
        ],
    )
    def k(faug_hbm, src_hbm, dst_hbm, out_hbm, src_a, dst_a, src_b, dst_b,
          rows_a, rows_b, acc_sh, sem_g, sem_s):
        cid = lax.axis_index("c")
        sid = lax.axis_index("s")
        wid = sid * NC + cid
        ebase = wid * EPT

        srcs = [src_a, src_b]
        dsts = [dst_a, dst_b]
        rows = [rows_a, rows_b]

        def zrow(r, carry):
            def zcol(q, c):
                rows_a[r, pl.ds(q * 16, 16)] = jnp.zeros((16,), jnp.float32)
                return c
            return lax.fori_loop(0, DA // 16, zcol, carry)
        lax.fori_loop(0, ZC, zrow, None)
        for j in range(ZPT):
            r0 = (sid + j * NS) * ZC
            pltpu.sync_copy(rows_a.at[pl.ds(0, ZC)], acc_sh.at[pl.ds(r0, ZC)])
        plsc.subcore_barrier()

        def load_idx(i, p):
            pltpu.sync_copy(src_hbm.at[pl.ds(ebase + i * CHUNK, CHUNK)],
                            srcs[p])
            pltpu.sync_copy(dst_hbm.at[pl.ds(ebase + i * CHUNK, CHUNK)],
                            dsts[p])

        def fire_g(p):
            pltpu.async_copy(faug_hbm.at[srcs[p]], rows[p], sem_g)

        def wait_g(p):
            pltpu.make_async_copy(faug_hbm.at[srcs[p]], rows[p], sem_g).wait()

        def fire_s(p):
            pltpu.async_copy(rows[p], acc_sh.at[dsts[p]], sem_s, add=True)

        def wait_s(p):
            pltpu.make_async_copy(rows[p], acc_sh.at[dsts[p]], sem_s).wait()

        def chunk(i, p, first=False, last=False):
            if not first:
                wait_s(1 - p)
            if not last:
                load_idx(i + 1, 1 - p)
            wait_g(p)
            if not last:
                fire_g(1 - p)
            fire_s(p)

        load_idx(0, 0)
        fire_g(0)
        chunk(0, 0, first=True)

        def pair(j, carry):
            i = 1 + 2 * j
            chunk(i, 1)
            chunk(i + 1, 0)
            return carry
        lax.fori_loop(0, (NCHUNK - 2) // 2, pair, None)

        chunk(NCHUNK - 1, (NCHUNK - 1) % 2, last=True)
        wait_s((NCHUNK - 1) % 2)
        plsc.subcore_barrier()

        for j in range(ZPT):
            r0 = (sid + j * NS) * ZC
            pltpu.sync_copy(acc_sh.at[pl.ds(r0, ZC)], rows_a.at[pl.ds(0, ZC)])
            pltpu.sync_copy(rows_a.at[pl.ds(0, ZC)], out_hbm.at[cid, pl.ds(r0, ZC)])

    return k(faug, src, dst)


def _tc_combine(feature, parts, W, b):
    def body(f_ref, p_ref, w_ref, b_ref, o_ref):
        a = p_ref[0, :N] + p_ref[1, :N]
        agg = a[:, :D]
        deg = jnp.sum(a[:, D:], axis=1, keepdims=True)
        neigh = agg / jnp.maximum(deg, 1.0)
        out = (
            jnp.dot(f_ref[...], w_ref[:D, :], preferred_element_type=jnp.float32)
            + jnp.dot(neigh, w_ref[D:, :], preferred_element_type=jnp.float32)
            + b_ref[...][None, :]
        )
        o_ref[...] = jnp.maximum(out, 0.0)

    return pl.pallas_call(
        body,
        out_shape=jax.ShapeDtypeStruct((N, D), jnp.float32),
    )(feature, parts, W, b)


def kernel(feature, edge_index, W, b):
    faug = jnp.concatenate(
        [feature,
         jnp.ones((N, 1), feature.dtype),
         jnp.zeros((N, DA - D - 1), feature.dtype)],
        axis=1,
    )
    pad = EP - E
    src = jnp.concatenate([edge_index[0], jnp.zeros((pad,), jnp.int32)])
    dst = jnp.concatenate([edge_index[1], jnp.full((pad,), N, jnp.int32)])
    parts = _sc_aggregate(faug, src, dst)
    return _tc_combine(feature, parts, W, b)


# speedup vs baseline: 1.5192x; 1.5192x over previous
import functools

import jax
import jax.numpy as jnp
from jax import lax
from jax.experimental import pallas as pl
from jax.experimental.pallas import tpu as pltpu
from jax.experimental.pallas import tpu_sc as plsc

N = 10000
E = 320000
D = 128
DA = 144
NP = 10240

NC = 2
NS = 16
NW = NC * NS
CHUNK = 80
NCHUNK = 125
EPT = NCHUNK * CHUNK
ZPT = NP // CHUNK // NS


def _sc_aggregate(faug, src, dst):
    mesh = plsc.VectorSubcoreMesh(core_axis_name="c", subcore_axis_name="s")

    @functools.partial(
        pl.kernel,
        mesh=mesh,
        compiler_params=pltpu.CompilerParams(use_tc_tiling_on_sc=False),
        out_type=jax.ShapeDtypeStruct((NC, NP, DA), jnp.float32),
        scratch_types=[
            pltpu.VMEM((CHUNK,), jnp.int32),
            pltpu.VMEM((CHUNK,), jnp.int32),
            pltpu.VMEM((CHUNK,), jnp.int32),
            pltpu.VMEM((CHUNK,), jnp.int32),
            pltpu.VMEM((CHUNK,), jnp.int32),
            pltpu.VMEM((CHUNK,), jnp.int32),
            pltpu.VMEM((CHUNK, DA), jnp.float32),
            pltpu.VMEM((CHUNK, DA), jnp.float32),
            pltpu.VMEM((CHUNK, DA), jnp.float32),
            pltpu.VMEM_SHARED((NP, DA), jnp.float32),
            pltpu.SemaphoreType.DMA,
            pltpu.SemaphoreType.DMA,
        ],
    )
    def k(faug_hbm, src_hbm, dst_hbm, out_hbm, src_a, dst_a, src_b, dst_b,
          src_c, dst_c, rows_a, rows_b, rows_c, acc_sh, sem_g, sem_s):
        cid = lax.axis_index("c")
        sid = lax.axis_index("s")
        wid = sid * NC + cid
        ebase = wid * EPT

        srcs = [src_a, src_b, src_c]
        dsts = [dst_a, dst_b, dst_c]
        rows = [rows_a, rows_b, rows_c]

        def zrow(r, carry):
            def zcol(q, c):
                rows_a[r, pl.ds(q * 16, 16)] = jnp.zeros((16,), jnp.float32)
                return c
            return lax.fori_loop(0, DA // 16, zcol, carry)
        lax.fori_loop(0, CHUNK, zrow, None)
        for j in range(ZPT):
            r0 = (sid + j * NS) * CHUNK
            pltpu.sync_copy(rows_a, acc_sh.at[pl.ds(r0, CHUNK)])
        plsc.subcore_barrier()

        def load_idx(i, p):
            pltpu.sync_copy(src_hbm.at[pl.ds(ebase + i * CHUNK, CHUNK)],
                            srcs[p])
            pltpu.sync_copy(dst_hbm.at[pl.ds(ebase + i * CHUNK, CHUNK)],
                            dsts[p])

        def fire_g(p):
            pltpu.async_copy(faug_hbm.at[srcs[p]], rows[p], sem_g)

        def wait_g(p):
            pltpu.make_async_copy(faug_hbm.at[srcs[p]], rows[p], sem_g).wait()

        def fire_s(p):
            pltpu.async_copy(rows[p], acc_sh.at[dsts[p]], sem_s, add=True)

        def wait_s(p):
            pltpu.make_async_copy(rows[p], acc_sh.at[dsts[p]], sem_s).wait()

        def slot(i, b, first=False, fire2=True):
            b2 = (b + 2) % 3
            wait_g(b)
            if not first:
                wait_s(b2)
            if fire2:
                load_idx(i + 2, b2)
                fire_g(b2)
            fire_s(b)

        load_idx(0, 0)
        load_idx(1, 1)
        fire_g(0)
        fire_g(1)
        slot(0, 0, first=True)

        def group(j, carry):
            i0 = 1 + 3 * j
            for t in range(3):
                slot(i0 + t, (1 + t) % 3)
            return carry
        lax.fori_loop(0, (NCHUNK - 5) // 3, group, None)

        slot(NCHUNK - 4, (NCHUNK - 4) % 3)
        slot(NCHUNK - 3, (NCHUNK - 3) % 3)
        slot(NCHUNK - 2, (NCHUNK - 2) % 3, fire2=False)
        slot(NCHUNK - 1, (NCHUNK - 1) % 3, fire2=False)
        wait_s((NCHUNK - 1) % 3)
        plsc.subcore_barrier()

        for j in range(ZPT):
            r0 = (sid + j * NS) * CHUNK
            pltpu.sync_copy(acc_sh.at[pl.ds(r0, CHUNK)], rows_a)
            pltpu.sync_copy(rows_a, out_hbm.at[cid, pl.ds(r0, CHUNK)])

    return k(faug, src, dst)


def _tc_combine(feature, parts, W, b):
    def body(f_ref, p_ref, w_ref, b_ref, o_ref):
        a = p_ref[0, :N] + p_ref[1, :N]
        agg = a[:, :D]
        deg = jnp.sum(a[:, D:], axis=1, keepdims=True)
        neigh = agg / jnp.maximum(deg, 1.0)
        out = (
            jnp.dot(f_ref[...], w_ref[:D, :], preferred_element_type=jnp.float32)
            + jnp.dot(neigh, w_ref[D:, :], preferred_element_type=jnp.float32)
            + b_ref[...][None, :]
        )
        o_ref[...] = jnp.maximum(out, 0.0)

    return pl.pallas_call(
        body,
        out_shape=jax.ShapeDtypeStruct((N, D), jnp.float32),
    )(feature, parts, W, b)


def kernel(feature, edge_index, W, b):
    faug = jnp.concatenate(
        [feature,
         jnp.ones((N, 1), feature.dtype),
         jnp.zeros((N, DA - D - 1), feature.dtype)],
        axis=1,
    )
    parts = _sc_aggregate(faug, edge_index[0], edge_index[1])
    return _tc_combine(feature, parts, W, b)
